# baseline (speedup 1.0000x reference)
import functools

import jax
import jax.numpy as jnp
from jax import lax
from jax.experimental import pallas as pl
from jax.experimental.pallas import tpu as pltpu
from jax.experimental.pallas import tpu_sc as plsc

N = 10000
E = 320000
D = 128
NC = 2
NS = 16
NW = NC * NS
CH = 80
CPT = 125
EPT = CPT * CH
PCH = 80
NPAD = 10240
RPT = NPAD // NS


def _make_sc_kernel(with_deg):
  mesh = plsc.VectorSubcoreMesh(core_axis_name="c", subcore_axis_name="s")
  agg_ty = jax.ShapeDtypeStruct((NC, NS, RPT, D), jnp.float32)
  out_type = (agg_ty, agg_ty) if with_deg else agg_ty
  NB = 3
  scratch = (
      [pltpu.VMEM((CH,), jnp.int32) for _ in range(NB)] +
      [pltpu.VMEM((CH,), jnp.int32) for _ in range(NB)] +
      [pltpu.VMEM((CH, D), jnp.float32) for _ in range(NB)] +
      [pltpu.VMEM_SHARED((NPAD, D), jnp.float32)] +
      [pltpu.SemaphoreType.DMA for _ in range(3 * NB)]
  )

  @functools.partial(pl.kernel, mesh=mesh, out_type=out_type,
                     scratch_types=scratch)
  def body(x_hbm, src_hbm, dst_hbm, zagg_hbm, ones_hbm, *rest):
    if with_deg:
      out_agg, out_deg, *rest = rest
    else:
      out_deg = None
      out_agg, *rest = rest
    srcs = rest[0:NB]
    dsts = rest[NB:2 * NB]
    rows = rest[2 * NB:3 * NB]
    agg_sh = rest[3 * NB]
    sems = rest[3 * NB + 1:3 * NB + 1 + NB]
    ssems = rest[3 * NB + 1 + NB:3 * NB + 1 + 2 * NB]
    dsems = rest[3 * NB + 1 + 2 * NB:3 * NB + 1 + 3 * NB]
    rows_v0, rows_v1 = rows[0], rows[1]
    c = lax.axis_index("c")
    s = lax.axis_index("s")
    tid = c * NS + s
    stripe = s * RPT
    base = tid * EPT

    def chunk_off(i):
      return pl.multiple_of(base + i * CH, 8)

    def zero_phase():
      pltpu.sync_copy(zagg_hbm, rows_v0)

      @pl.loop(0, RPT // PCH)
      def zstep(j):
        pltpu.async_copy(rows_v0.at[pl.ds(0, PCH)],
                         agg_sh.at[pl.ds(stripe + j * PCH, PCH)], sems[2])

      @pl.loop(0, RPT // PCH)
      def zwait(j):
        pltpu.make_async_copy(
            rows_v0.at[pl.ds(0, PCH)],
            agg_sh.at[pl.ds(stripe, PCH)], sems[2]).wait()

    def publish_phase(out):
      @pl.loop(0, RPT // (2 * PCH))
      def pstep(j):
        pltpu.sync_copy(agg_sh.at[pl.ds(stripe + (2 * j) * PCH, PCH)],
                        rows_v0.at[pl.ds(0, PCH)])
        pltpu.async_copy(rows_v0.at[pl.ds(0, PCH)],
                         out.at[c, s, pl.ds((2 * j) * PCH, PCH)], sems[0])
        pltpu.sync_copy(agg_sh.at[pl.ds(stripe + (2 * j + 1) * PCH, PCH)],
                        rows_v1.at[pl.ds(0, PCH)])
        pltpu.async_copy(rows_v1.at[pl.ds(0, PCH)],
                         out.at[c, s, pl.ds((2 * j + 1) * PCH, PCH)],
                         sems[1])
        pltpu.make_async_copy(
            rows_v0.at[pl.ds(0, PCH)],
            out.at[c, s, pl.ds((2 * j) * PCH, PCH)], sems[0]).wait()
        pltpu.make_async_copy(
            rows_v1.at[pl.ds(0, PCH)],
            out.at[c, s, pl.ds((2 * j + 1) * PCH, PCH)], sems[1]).wait()

    def prefetch(i, b):
      i = jnp.where(i < CPT, i, 0)
      off = chunk_off(i)
      pltpu.async_copy(dst_hbm.at[pl.ds(off, CH)], dsts[b], dsems[b])
      pltpu.async_copy(src_hbm.at[pl.ds(off, CH)], srcs[b], ssems[b])

    def fire(b):
      pltpu.make_async_copy(src_hbm.at[pl.ds(0, CH)], srcs[b],
                            ssems[b]).wait()
      pltpu.async_copy(x_hbm.at[srcs[b]], rows[b], sems[b])

    def drain(b):
      pltpu.make_async_copy(x_hbm.at[srcs[b]], rows[b], sems[b]).wait()
      pltpu.make_async_copy(dst_hbm.at[pl.ds(0, CH)], dsts[b],
                            dsems[b]).wait()
      pltpu.sync_copy(rows[b], agg_sh.at[dsts[b]], add=True)

    zero_phase()
    plsc.subcore_barrier()
    prefetch(0, 0)
    prefetch(1, 1)
    fire(0)
    fire(1)
    prefetch(2, 2)

    @pl.loop(0, CPT - 2, step=3)
    def step(i):
      fire(2)
      drain(0)
      prefetch(i + 3, 0)
      drain(1)
      prefetch(i + 4, 1)
      fire(0)
      drain(2)
      prefetch(i + 5, 2)
      fire(1)

    drain(0)
    drain(1)
    pltpu.make_async_copy(src_hbm.at[pl.ds(0, CH)], srcs[2], ssems[2]).wait()
    pltpu.make_async_copy(dst_hbm.at[pl.ds(0, CH)], dsts[2], dsems[2]).wait()
    plsc.subcore_barrier()
    publish_phase(out_agg)

    if with_deg:
      plsc.subcore_barrier()
      zero_phase()
      pltpu.sync_copy(zagg_hbm, rows_v1)
      for b in range(NB):
        pltpu.async_copy(rows_v1, agg_sh.at[dsts[b]], ssems[b], add=True)
      pltpu.sync_copy(ones_hbm, rows_v0)

      def dfetch(i, b):
        pltpu.make_async_copy(rows_v0, agg_sh.at[dsts[b]],
                              ssems[b]).wait()
        pltpu.async_copy(dst_hbm.at[pl.ds(chunk_off(i), CH)], dsts[b],
                         sems[b])

      def ddrain(b):
        pltpu.make_async_copy(dst_hbm.at[pl.ds(0, CH)], dsts[b],
                              sems[b]).wait()
        pltpu.async_copy(rows_v0, agg_sh.at[dsts[b]], ssems[b], add=True)

      plsc.subcore_barrier()
      dfetch(0, 0)
      dfetch(1, 1)

      @pl.loop(0, CPT - 2, step=3)
      def dstep(i):
        dfetch(i + 2, 2)
        ddrain(0)
        dfetch(i + 3, 0)
        ddrain(1)
        dfetch(i + 4, 1)
        ddrain(2)

      ddrain(0)
      ddrain(1)
      for b in range(NB):
        pltpu.make_async_copy(rows_v0, agg_sh.at[dsts[b]],
                              ssems[b]).wait()
      plsc.subcore_barrier()
      publish_phase(out_deg)

  return body


_sc_agg_deg = _make_sc_kernel(True)
_sc_agg = _make_sc_kernel(False)


def _tc_combine(p_agg, p_deg, x, W, b2d, Wr):
  BN = 1024

  def body(pa_ref, pd_ref, x_ref, w_ref, b_ref, wr_ref, o_ref):
    agg = pa_ref[0] + pa_ref[1]
    deg = pd_ref[0, :, :1] + pd_ref[1, :, :1]
    agg = agg * (1.0 / jnp.maximum(deg, 1.0))
    h = jnp.dot(agg, w_ref[...], preferred_element_type=jnp.float32)
    h = h + b_ref[...]
    r = jnp.dot(x_ref[...], wr_ref[...], preferred_element_type=jnp.float32)
    o_ref[...] = jnp.maximum(h, 0.0) + r

  return pl.pallas_call(
      body,
      grid=(NPAD // BN,),
      in_specs=[
          pl.BlockSpec((NC, BN, D), lambda i: (0, i, 0)),
          pl.BlockSpec((NC, BN, D), lambda i: (0, i, 0)),
          pl.BlockSpec((BN, D), lambda i: (i, 0)),
          pl.BlockSpec((D, D), lambda i: (0, 0)),
          pl.BlockSpec((1, D), lambda i: (0, 0)),
          pl.BlockSpec((D, D), lambda i: (0, 0)),
      ],
      out_specs=pl.BlockSpec((BN, D), lambda i: (i, 0)),
      out_shape=jax.ShapeDtypeStruct((NPAD, D), jnp.float32),
  )(p_agg, p_deg, x, W, b2d, Wr)


def kernel(x, edge_index, W1, b1, Wr1, W2, b2, Wr2):
  ept_real = E // NW
  e0 = edge_index[0].astype(jnp.int32).reshape(NW, ept_real)
  e1 = edge_index[1].astype(jnp.int32).reshape(NW, ept_real)
  pad_dst = N + (jnp.arange(EPT - ept_real, dtype=jnp.int32) % (NPAD - N))
  src = jnp.concatenate(
      [e0, jnp.zeros((NW, EPT - ept_real), jnp.int32)], axis=1).reshape(-1)
  dst = jnp.concatenate(
      [e1, jnp.broadcast_to(pad_dst, (NW, EPT - ept_real))], axis=1
  ).reshape(-1)
  zagg = jnp.zeros((CH, D), jnp.float32)
  ones_blk = jnp.ones((CH, D), jnp.float32)
  xp = jnp.zeros((NPAD, D), jnp.float32).at[:N].set(x)

  pa1, pdeg = _sc_agg_deg(xp, src, dst, zagg, ones_blk)
  pa1 = pa1.reshape(NC, NPAD, D)
  pdeg = pdeg.reshape(NC, NPAD, D)
  h = _tc_combine(pa1, pdeg, xp, W1, b1.reshape(1, D), Wr1)
  pa2 = _sc_agg(h, src, dst, zagg, ones_blk).reshape(NC, NPAD, D)
  out = _tc_combine(pa2, pdeg, h, W2, b2.reshape(1, D), Wr2)
  return out[:N]


# speedup vs baseline: 1.0045x; 1.0045x over previous
import functools

import jax
import jax.numpy as jnp
from jax import lax
from jax.experimental import pallas as pl
from jax.experimental.pallas import tpu as pltpu
from jax.experimental.pallas import tpu_sc as plsc

N = 10000
E = 320000
D = 128
NC = 2
NS = 16
NW = NC * NS
CH = 80
CPT = 125
EPT = CPT * CH
PCH = 80
NPAD = 10240
RPT = NPAD // NS


def _make_sc_kernel(with_deg):
  mesh = plsc.VectorSubcoreMesh(core_axis_name="c", subcore_axis_name="s")
  agg_ty = jax.ShapeDtypeStruct((NC, NS, RPT, D), jnp.float32)
  out_type = (agg_ty, agg_ty) if with_deg else agg_ty
  NB = 3
  scratch = (
      [pltpu.VMEM((CH,), jnp.int32) for _ in range(NB)] +
      [pltpu.VMEM((CH,), jnp.int32) for _ in range(NB)] +
      [pltpu.VMEM((CH, D), jnp.float32) for _ in range(NB)] +
      [pltpu.VMEM_SHARED((NPAD, D), jnp.float32)] +
      [pltpu.SemaphoreType.DMA for _ in range(3 * NB)]
  )

  @functools.partial(pl.kernel, mesh=mesh, out_type=out_type,
                     scratch_types=scratch)
  def body(x_hbm, src_hbm, dst_hbm, zagg_hbm, ones_hbm, *rest):
    if with_deg:
      out_agg, out_deg, *rest = rest
    else:
      out_deg = None
      out_agg, *rest = rest
    srcs = rest[0:NB]
    dsts = rest[NB:2 * NB]
    rows = rest[2 * NB:3 * NB]
    agg_sh = rest[3 * NB]
    sems = rest[3 * NB + 1:3 * NB + 1 + NB]
    ssems = rest[3 * NB + 1 + NB:3 * NB + 1 + 2 * NB]
    dsems = rest[3 * NB + 1 + 2 * NB:3 * NB + 1 + 3 * NB]
    rows_v0, rows_v1 = rows[0], rows[1]
    c = lax.axis_index("c")
    s = lax.axis_index("s")
    tid = c * NS + s
    stripe = s * RPT
    base = tid * EPT

    def chunk_off(i):
      return pl.multiple_of(base + i * CH, 8)

    def zero_phase():
      pltpu.sync_copy(zagg_hbm, rows_v0)

      @pl.loop(0, RPT // PCH)
      def zstep(j):
        pltpu.async_copy(rows_v0.at[pl.ds(0, PCH)],
                         agg_sh.at[pl.ds(stripe + j * PCH, PCH)], sems[2])

      @pl.loop(0, RPT // PCH)
      def zwait(j):
        pltpu.make_async_copy(
            rows_v0.at[pl.ds(0, PCH)],
            agg_sh.at[pl.ds(stripe, PCH)], sems[2]).wait()

    def publish_phase(out):
      @pl.loop(0, RPT // (2 * PCH))
      def pstep(j):
        pltpu.sync_copy(agg_sh.at[pl.ds(stripe + (2 * j) * PCH, PCH)],
                        rows_v0.at[pl.ds(0, PCH)])
        pltpu.async_copy(rows_v0.at[pl.ds(0, PCH)],
                         out.at[c, s, pl.ds((2 * j) * PCH, PCH)], sems[0])
        pltpu.sync_copy(agg_sh.at[pl.ds(stripe + (2 * j + 1) * PCH, PCH)],
                        rows_v1.at[pl.ds(0, PCH)])
        pltpu.async_copy(rows_v1.at[pl.ds(0, PCH)],
                         out.at[c, s, pl.ds((2 * j + 1) * PCH, PCH)],
                         sems[1])
        pltpu.make_async_copy(
            rows_v0.at[pl.ds(0, PCH)],
            out.at[c, s, pl.ds((2 * j) * PCH, PCH)], sems[0]).wait()
        pltpu.make_async_copy(
            rows_v1.at[pl.ds(0, PCH)],
            out.at[c, s, pl.ds((2 * j + 1) * PCH, PCH)], sems[1]).wait()

    def prefetch(i, b):
      i = jnp.where(i < CPT, i, 0)
      off = chunk_off(i)
      pltpu.async_copy(dst_hbm.at[pl.ds(off, CH)], dsts[b], dsems[b])
      pltpu.async_copy(src_hbm.at[pl.ds(off, CH)], srcs[b], ssems[b])

    def fire(b):
      pltpu.make_async_copy(src_hbm.at[pl.ds(0, CH)], srcs[b],
                            ssems[b]).wait()
      pltpu.async_copy(x_hbm.at[srcs[b]], rows[b], sems[b])

    def drain(b):
      pltpu.make_async_copy(x_hbm.at[srcs[b]], rows[b], sems[b]).wait()
      pltpu.make_async_copy(dst_hbm.at[pl.ds(0, CH)], dsts[b],
                            dsems[b]).wait()
      pltpu.sync_copy(rows[b], agg_sh.at[dsts[b]], add=True)

    zero_phase()
    plsc.subcore_barrier()
    prefetch(0, 0)
    prefetch(1, 1)
    fire(0)
    fire(1)
    prefetch(2, 2)

    @pl.loop(0, CPT - 2, step=3)
    def step(i):
      fire(2)
      drain(0)
      prefetch(i + 3, 0)
      drain(1)
      prefetch(i + 4, 1)
      fire(0)
      drain(2)
      prefetch(i + 5, 2)
      fire(1)

    drain(0)
    drain(1)
    pltpu.make_async_copy(src_hbm.at[pl.ds(0, CH)], srcs[2], ssems[2]).wait()
    pltpu.make_async_copy(dst_hbm.at[pl.ds(0, CH)], dsts[2], dsems[2]).wait()
    plsc.subcore_barrier()
    publish_phase(out_agg)

    if with_deg:
      plsc.subcore_barrier()
      zero_phase()
      pltpu.sync_copy(ones_hbm, rows_v0)

      def dfetch(i, b):
        pltpu.async_copy(dst_hbm.at[pl.ds(chunk_off(i), CH)], dsts[b],
                         sems[b])

      def ddrain(b):
        pltpu.make_async_copy(dst_hbm.at[pl.ds(0, CH)], dsts[b],
                              sems[b]).wait()
        pltpu.sync_copy(rows_v0, agg_sh.at[dsts[b]], add=True)

      plsc.subcore_barrier()
      dfetch(0, 0)
      dfetch(1, 1)

      @pl.loop(0, CPT - 2, step=3)
      def dstep(i):
        dfetch(i + 2, 2)
        ddrain(0)
        dfetch(i + 3, 0)
        ddrain(1)
        dfetch(i + 4, 1)
        ddrain(2)

      ddrain(0)
      ddrain(1)
      plsc.subcore_barrier()
      publish_phase(out_deg)

  return body


_sc_agg_deg = _make_sc_kernel(True)
_sc_agg = _make_sc_kernel(False)


def _tc_combine(p_agg, p_deg, x, W, b2d, Wr):
  BN = 1024

  def body(pa_ref, pd_ref, x_ref, w_ref, b_ref, wr_ref, o_ref):
    agg = pa_ref[0] + pa_ref[1]
    deg = pd_ref[0, :, :1] + pd_ref[1, :, :1]
    agg = agg * (1.0 / jnp.maximum(deg, 1.0))
    h = jnp.dot(agg, w_ref[...], preferred_element_type=jnp.float32)
    h = h + b_ref[...]
    r = jnp.dot(x_ref[...], wr_ref[...], preferred_element_type=jnp.float32)
    o_ref[...] = jnp.maximum(h, 0.0) + r

  return pl.pallas_call(
      body,
      grid=(NPAD // BN,),
      in_specs=[
          pl.BlockSpec((NC, BN, D), lambda i: (0, i, 0)),
          pl.BlockSpec((NC, BN, D), lambda i: (0, i, 0)),
          pl.BlockSpec((BN, D), lambda i: (i, 0)),
          pl.BlockSpec((D, D), lambda i: (0, 0)),
          pl.BlockSpec((1, D), lambda i: (0, 0)),
          pl.BlockSpec((D, D), lambda i: (0, 0)),
      ],
      out_specs=pl.BlockSpec((BN, D), lambda i: (i, 0)),
      out_shape=jax.ShapeDtypeStruct((NPAD, D), jnp.float32),
  )(p_agg, p_deg, x, W, b2d, Wr)


def kernel(x, edge_index, W1, b1, Wr1, W2, b2, Wr2):
  ept_real = E // NW
  e0 = edge_index[0].astype(jnp.int32).reshape(NW, ept_real)
  e1 = edge_index[1].astype(jnp.int32).reshape(NW, ept_real)
  pad_dst = N + (jnp.arange(EPT - ept_real, dtype=jnp.int32) % (NPAD - N))
  src = jnp.concatenate(
      [e0, jnp.zeros((NW, EPT - ept_real), jnp.int32)], axis=1).reshape(-1)
  dst = jnp.concatenate(
      [e1, jnp.broadcast_to(pad_dst, (NW, EPT - ept_real))], axis=1
  ).reshape(-1)
  zagg = jnp.zeros((CH, D), jnp.float32)
  ones_blk = jnp.ones((CH, D), jnp.float32)
  xp = jnp.zeros((NPAD, D), jnp.float32).at[:N].set(x)

  pa1, pdeg = _sc_agg_deg(xp, src, dst, zagg, ones_blk)
  pa1 = pa1.reshape(NC, NPAD, D)
  pdeg = pdeg.reshape(NC, NPAD, D)
  h = _tc_combine(pa1, pdeg, xp, W1, b1.reshape(1, D), Wr1)
  pa2 = _sc_agg(h, src, dst, zagg, ones_blk).reshape(NC, NPAD, D)
  out = _tc_combine(pa2, pdeg, h, W2, b2.reshape(1, D), Wr2)
  return out[:N]
